# baseline (speedup 1.0000x reference)
import jax
import jax.numpy as jnp
from jax.experimental import pallas as pl
from jax.experimental.pallas import tpu as pltpu

NUM_EMB = 1024
DIM = 64
COMMIT = 0.25
N_ROWS = 16384
NBLK = 16
BR = N_ROWS // NBLK


def _vq_body(x_ref, eaug_ref, e2_ref, enc_ref, q_ref, loss_ref, perp_ref,
             esq_ref, counts_ref, loss_acc):
    i = pl.program_id(0)

    @pl.when(i == 0)
    def _():
        e = jnp.float32(-0.5) * e2_ref[...]
        esq_ref[...] = jnp.sum(e * e, axis=1)[None, :]
        counts_ref[...] = jnp.zeros((1, NUM_EMB), jnp.float32)
        loss_acc[0] = jnp.float32(0.0)

    x = x_ref[...]
    xsq = jnp.sum(x * x, axis=1, keepdims=True)
    xe2 = jax.lax.dot_general(x, e2_ref[...], (((1,), (1,)), ((), ())),
                              preferred_element_type=jnp.float32)
    d = (xsq + esq_ref[...]) + xe2
    m = jnp.min(d, axis=1, keepdims=True)
    enc = jnp.where(d == m, 1.0, 0.0).astype(jnp.float32)
    enc_ref[...] = enc
    q65 = jax.lax.dot_general(enc.astype(jnp.bfloat16), eaug_ref[...],
                              (((1,), (0,)), ((), ())),
                              preferred_element_type=jnp.float32)
    q_ref[...] = q65[:, :DIM]
    loss_acc[0] += jnp.sum(m)
    tie = jnp.max(q65[:, DIM:DIM + 1]) != jnp.float32(1.0)

    @pl.when(jnp.logical_not(tie))
    def _():
        counts_ref[...] += jnp.sum(enc, axis=0, keepdims=True)

    @pl.when(tie)
    def _():
        lane = jax.lax.broadcasted_iota(jnp.int32, (BR, NUM_EMB), 1)
        masked = jnp.where(d == m, lane, NUM_EMB)
        idx = jnp.min(masked, axis=1, keepdims=True)
        enc2 = jnp.where(lane == idx, 1.0, 0.0).astype(jnp.float32)
        enc_ref[...] = enc2
        q65b = jax.lax.dot_general(enc2.astype(jnp.bfloat16), eaug_ref[...],
                                   (((1,), (0,)), ((), ())),
                                   preferred_element_type=jnp.float32)
        q_ref[...] = q65b[:, :DIM]
        counts_ref[...] += jnp.sum(enc2, axis=0, keepdims=True)

    @pl.when(i == NBLK - 1)
    def _():
        loss_ref[0, 0] = loss_acc[0] * (COMMIT / (N_ROWS * DIM))
        probs = counts_ref[...] * (1.0 / N_ROWS)
        ent = -jnp.sum(probs * jnp.log(probs + 1e-10))
        perp_ref[0, 0] = jnp.exp(ent)


def kernel(inputs, embedding_weight):
    B, C, H, W = inputs.shape
    flat = jnp.transpose(inputs, (0, 2, 3, 1)).reshape(-1, C)
    eaug = jnp.concatenate(
        [embedding_weight, jnp.ones((NUM_EMB, 1), jnp.float32)],
        axis=1).astype(jnp.bfloat16)
    e2 = -2.0 * embedding_weight
    enc, q, loss, perp = pl.pallas_call(
        _vq_body,
        grid=(NBLK,),
        in_specs=[
            pl.BlockSpec((BR, DIM), lambda i: (i, 0)),
            pl.BlockSpec((NUM_EMB, DIM + 1), lambda i: (0, 0)),
            pl.BlockSpec((NUM_EMB, DIM), lambda i: (0, 0)),
        ],
        out_specs=[
            pl.BlockSpec((BR, NUM_EMB), lambda i: (i, 0)),
            pl.BlockSpec((BR, DIM), lambda i: (i, 0)),
            pl.BlockSpec(memory_space=pltpu.SMEM),
            pl.BlockSpec(memory_space=pltpu.SMEM),
        ],
        out_shape=[
            jax.ShapeDtypeStruct((N_ROWS, NUM_EMB), jnp.float32),
            jax.ShapeDtypeStruct((N_ROWS, DIM), jnp.float32),
            jax.ShapeDtypeStruct((1, 1), jnp.float32),
            jax.ShapeDtypeStruct((1, 1), jnp.float32),
        ],
        scratch_shapes=[
            pltpu.VMEM((1, NUM_EMB), jnp.float32),
            pltpu.VMEM((1, NUM_EMB), jnp.float32),
            pltpu.SMEM((1,), jnp.float32),
        ],
        compiler_params=pltpu.CompilerParams(
            dimension_semantics=("arbitrary",)),
    )(flat, eaug, e2)
    q_out = jnp.transpose(q.reshape(B, H, W, C), (0, 3, 1, 2))
    return loss[0, 0], q_out, perp[0, 0], enc


# speedup vs baseline: 1.1147x; 1.1147x over previous
import jax
import jax.numpy as jnp
from jax.experimental import pallas as pl
from jax.experimental.pallas import tpu as pltpu

NUM_EMB = 1024
DIM = 64
COMMIT = 0.25
N_ROWS = 16384
NBLK = 8
BR = N_ROWS // NBLK


def _vq_body(x_ref, eaug_ref, e2_ref, enc_ref, q_ref, loss_ref, perp_ref,
             esq_ref, counts_ref, loss_acc):
    i = pl.program_id(0)

    @pl.when(i == 0)
    def _():
        e = eaug_ref[:, :DIM]
        esq_ref[...] = jnp.sum(e * e, axis=1)[None, :]
        counts_ref[...] = jnp.zeros((1, NUM_EMB), jnp.float32)
        loss_acc[0] = jnp.float32(0.0)

    x = x_ref[...]
    xsq = jnp.sum(x * x, axis=1, keepdims=True)
    xe2 = jax.lax.dot_general(x, e2_ref[...], (((1,), (1,)), ((), ())),
                              preferred_element_type=jnp.float32)
    d = (xsq + esq_ref[...]) + xe2
    m = jnp.min(d, axis=1, keepdims=True)
    enc = jnp.where(d == m, 1.0, 0.0).astype(jnp.float32)
    enc_ref[...] = enc
    q65 = jax.lax.dot_general(enc, eaug_ref[...], (((1,), (0,)), ((), ())),
                              preferred_element_type=jnp.float32)
    q_ref[...] = q65[:, :DIM]
    loss_acc[0] += jnp.sum(m)
    tie = jnp.max(q65[:, DIM:DIM + 1]) != jnp.float32(1.0)

    @pl.when(jnp.logical_not(tie))
    def _():
        counts_ref[...] += jnp.sum(enc, axis=0, keepdims=True)

    @pl.when(tie)
    def _():
        lane = jax.lax.broadcasted_iota(jnp.int32, (BR, NUM_EMB), 1)
        masked = jnp.where(d == m, lane, NUM_EMB)
        idx = jnp.min(masked, axis=1, keepdims=True)
        enc2 = jnp.where(lane == idx, 1.0, 0.0).astype(jnp.float32)
        enc_ref[...] = enc2
        q65b = jax.lax.dot_general(enc2, eaug_ref[...],
                                   (((1,), (0,)), ((), ())),
                                   preferred_element_type=jnp.float32)
        q_ref[...] = q65b[:, :DIM]
        counts_ref[...] += jnp.sum(enc2, axis=0, keepdims=True)

    @pl.when(i == NBLK - 1)
    def _():
        loss_ref[0, 0] = loss_acc[0] * (COMMIT / (N_ROWS * DIM))
        probs = counts_ref[...] * (1.0 / N_ROWS)
        ent = -jnp.sum(probs * jnp.log(probs + 1e-10))
        perp_ref[0, 0] = jnp.exp(ent)


def kernel(inputs, embedding_weight):
    B, C, H, W = inputs.shape
    flat = jnp.transpose(inputs, (0, 2, 3, 1)).reshape(-1, C)
    eaug = jnp.concatenate(
        [embedding_weight, jnp.ones((NUM_EMB, 1), jnp.float32)], axis=1)
    e2 = -2.0 * embedding_weight
    enc, q, loss, perp = pl.pallas_call(
        _vq_body,
        grid=(NBLK,),
        in_specs=[
            pl.BlockSpec((BR, DIM), lambda i: (i, 0)),
            pl.BlockSpec((NUM_EMB, DIM + 1), lambda i: (0, 0)),
            pl.BlockSpec((NUM_EMB, DIM), lambda i: (0, 0)),
        ],
        out_specs=[
            pl.BlockSpec((BR, NUM_EMB), lambda i: (i, 0)),
            pl.BlockSpec((BR, DIM), lambda i: (i, 0)),
            pl.BlockSpec(memory_space=pltpu.SMEM),
            pl.BlockSpec(memory_space=pltpu.SMEM),
        ],
        out_shape=[
            jax.ShapeDtypeStruct((N_ROWS, NUM_EMB), jnp.float32),
            jax.ShapeDtypeStruct((N_ROWS, DIM), jnp.float32),
            jax.ShapeDtypeStruct((1, 1), jnp.float32),
            jax.ShapeDtypeStruct((1, 1), jnp.float32),
        ],
        scratch_shapes=[
            pltpu.VMEM((1, NUM_EMB), jnp.float32),
            pltpu.VMEM((1, NUM_EMB), jnp.float32),
            pltpu.SMEM((1,), jnp.float32),
        ],
        compiler_params=pltpu.CompilerParams(
            dimension_semantics=("arbitrary",)),
    )(flat, eaug, e2)
    q_out = jnp.transpose(q.reshape(B, H, W, C), (0, 3, 1, 2))
    return loss[0, 0], q_out, perp[0, 0], enc
